# baseline (speedup 1.0000x reference)
import functools
from math import sqrt

import jax
import jax.numpy as jnp
from jax import lax
from jax.experimental import pallas as pl
from jax.experimental.pallas import tpu as pltpu
from jax.experimental.pallas import tpu_sc as plsc

D_MODEL = 64
SCALE = float(sqrt(D_MODEL))

NC = 2
NS = 16
NW = NC * NS
LANES = 16

CHUNK = 256
VECS_PER_ROW = D_MODEL // LANES


def _make_lookup(B, Vp):
    assert B % NW == 0
    b_per_w = B // NW
    assert b_per_w % CHUNK == 0
    nchunks = b_per_w // CHUNK

    mesh = plsc.VectorSubcoreMesh(
        core_axis_name="c", subcore_axis_name="s",
        num_cores=NC, num_subcores=NS)

    @functools.partial(
        pl.kernel,
        mesh=mesh,
        out_type=jax.ShapeDtypeStruct((B, D_MODEL), jnp.float32),
        scratch_types=[
            pltpu.VMEM((b_per_w,), jnp.int32),
            pltpu.VMEM((CHUNK,), jnp.int32),
            pltpu.VMEM((CHUNK,), jnp.int32),
            pltpu.VMEM((CHUNK, 2 * D_MODEL), jnp.float32),
            pltpu.VMEM((CHUNK, 2 * D_MODEL), jnp.float32),
            pltpu.VMEM((CHUNK, D_MODEL), jnp.float32),
            pltpu.SemaphoreType.DMA,
            pltpu.SemaphoreType.DMA,
        ],
    )
    def lookup(x_hbm, table_hbm, out_hbm, idx_v, pidx0, pidx1, buf0, buf1,
               obuf, sem0, sem1):
        wid = lax.axis_index("s") * NC + lax.axis_index("c")
        base = wid * b_per_w

        pltpu.sync_copy(x_hbm.at[pl.ds(base, b_per_w)], idx_v)

        bufs = (buf0, buf1)
        pidxs = (pidx0, pidx1)
        sems = (sem0, sem1)

        def compute_pairs(g, b):
            @pl.loop(0, CHUNK // LANES)
            def _pairs(i):
                v = idx_v[pl.ds(g * CHUNK + i * LANES, LANES)]
                pidxs[b][pl.ds(i * LANES, LANES)] = lax.shift_right_logical(
                    v, 1)

        def start_gather(g, b):
            pltpu.async_copy(table_hbm.at[pidxs[b]], bufs[b], sems[b])

        def scale_and_store(g, b):
            buf = bufs[b]
            pltpu.make_async_copy(table_hbm.at[pidxs[b]], buf,
                                  sems[b]).wait()

            @pl.loop(0, CHUNK // LANES)
            def _scale(i):
                hv = (idx_v[pl.ds(g * CHUNK + i * LANES, LANES)] & 1) * D_MODEL
                for j in range(LANES):
                    h = hv[j]
                    r = i * LANES + j
                    for k in range(VECS_PER_ROW):
                        obuf[r, pl.ds(k * LANES, LANES)] = (
                            buf[r, pl.ds(h + k * LANES, LANES)] * SCALE)

            pltpu.sync_copy(obuf, out_hbm.at[pl.ds(base + g * CHUNK, CHUNK)])

        compute_pairs(0, 0)
        start_gather(0, 0)
        compute_pairs(1, 1)
        start_gather(1, 1)

        @pl.loop(0, nchunks - 2, step=2)
        def _chunks(g0):
            for b in range(2):
                g = g0 + b
                scale_and_store(g, b)
                compute_pairs(g + 2, b)
                start_gather(g + 2, b)

        for b in range(2):
            scale_and_store(nchunks - 2 + b, b)

    return lookup


def kernel(X, table):
    rows, cols = X.shape
    B = rows * cols
    V = table.shape[0]
    xf = X.reshape(B).astype(jnp.int32)
    tresh = table.reshape(V // 2, 2 * D_MODEL)
    out = _make_lookup(B, V // 2)(xf, tresh)
    return out.reshape(rows, cols, D_MODEL)
